# baseline (speedup 1.0000x reference)
import dataclasses
import functools
import math

import jax
import jax.numpy as jnp
from jax import lax
from jax.experimental import pallas as pl
from jax.experimental.pallas import tpu as pltpu
from jax.experimental.pallas import tpu_sc as plsc

NUM_CORES = 2
NUM_SUBCORES = 16
CHUNK = 64
LANES = 16
BLK_CH = 16
NBUF = 4


def _sc_segment_sums(x_flat, idx_all, idx_flat, n_dst, chunks, z_rows):
    d = x_flat.shape[1]
    zero_base = x_flat.shape[0] - z_rows
    sp_rows = NUM_SUBCORES * z_rows
    blocks = chunks // BLK_CH

    mesh = plsc.VectorSubcoreMesh(core_axis_name="c", subcore_axis_name="s")

    cp = pltpu.CompilerParams()
    if "needs_layout_passes" in pltpu.CompilerParams.__dataclass_fields__:
        cp = dataclasses.replace(cp, needs_layout_passes=False)

    @functools.partial(
        pl.kernel,
        compiler_params=cp,
        out_type=[
            jax.ShapeDtypeStruct((2, sp_rows, d), jnp.float32),
            jax.ShapeDtypeStruct((2, NUM_SUBCORES, sp_rows), jnp.float32),
        ],
        mesh=mesh,
        scratch_types=[
            pltpu.VMEM((2, BLK_CH, CHUNK), jnp.int32),
            pltpu.VMEM((BLK_CH * CHUNK,), jnp.int32),
            pltpu.VMEM((NBUF, CHUNK, d), jnp.float32),
            pltpu.VMEM((sp_rows,), jnp.float32),
            pltpu.VMEM_SHARED((sp_rows, d), jnp.float32),
            pltpu.SemaphoreType.DMA,
            pltpu.SemaphoreType.DMA,
            pltpu.SemaphoreType.DMA,
            pltpu.SemaphoreType.DMA,
        ],
    )
    def k(x_hbm, idx_hbm, idxf_hbm, agg_hbm, cnt_hbm,
          idx_v, dst_f, rows, cnt_l, agg_sh, sem0, sem1, sem2, sem3):
        c = lax.axis_index("c")
        s = lax.axis_index("s")
        sems = [sem0, sem1, sem2, sem3]

        @pl.loop(0, sp_rows // LANES)
        def _(i):
            cnt_l[pl.ds(i * LANES, LANES)] = jnp.zeros((LANES,), jnp.float32)

        pltpu.sync_copy(x_hbm.at[pl.ds(zero_base, z_rows)],
                        agg_sh.at[pl.ds(s * z_rows, z_rows)])
        plsc.subcore_barrier()

        ones = jnp.full((LANES,), 1.0, jnp.float32)

        @pl.loop(0, blocks)
        def _(b):
            pltpu.sync_copy(
                idx_hbm.at[c].at[s].at[:, pl.ds(b * BLK_CH, BLK_CH), :],
                idx_v)
            pltpu.sync_copy(
                idxf_hbm.at[c].at[s].at[1].at[
                    pl.ds(b * BLK_CH * CHUNK, BLK_CH * CHUNK)],
                dst_f)

            for t in range(NBUF):
                pltpu.async_copy(x_hbm.at[idx_v.at[0].at[t]], rows.at[t],
                                 sems[t])

            @pl.loop(0, BLK_CH // NBUF)
            def _(g):
                for t in range(NBUF):
                    j = g * NBUF + t
                    pltpu.make_async_copy(x_hbm.at[idx_v.at[0].at[j]],
                                          rows.at[t], sems[t]).wait()
                    pltpu.sync_copy(rows.at[t],
                                    agg_sh.at[idx_v.at[1].at[j]], add=True)
                    for l in range(CHUNK // LANES):
                        idx16 = dst_f[pl.ds(j * CHUNK + l * LANES, LANES)]
                        plsc.addupdate_scatter(cnt_l, [idx16], ones)
                    @pl.when(g < BLK_CH // NBUF - 1)
                    def _():
                        pltpu.async_copy(
                            x_hbm.at[idx_v.at[0].at[j + NBUF]],
                            rows.at[t], sems[t])

        plsc.subcore_barrier()

        pltpu.sync_copy(agg_sh.at[pl.ds(s * z_rows, z_rows)],
                        agg_hbm.at[c].at[pl.ds(s * z_rows, z_rows)])
        pltpu.sync_copy(cnt_l, cnt_hbm.at[c].at[s])

    return k(x_flat, idx_all, idx_flat)


def _tc_combine(agg, cnt, x_paper,
                W_pp_nbr, W_ap_nbr, W_pp_self, W_ap_self, b_pp, b_ap):
    n, d = x_paper.shape
    blk = 1024
    grid = (math.ceil(n / blk),)
    agg_spec = pl.BlockSpec((1, blk, d), lambda i: (0, i, 0))
    agg_spec2 = pl.BlockSpec((1, blk, d), lambda i: (1, i, 0))
    cnt_spec = pl.BlockSpec((1, NUM_SUBCORES, blk), lambda i: (0, 0, i))
    cnt_spec2 = pl.BlockSpec((1, NUM_SUBCORES, blk), lambda i: (1, 0, i))
    row_spec = pl.BlockSpec((blk, d), lambda i: (i, 0))
    w_spec = pl.BlockSpec((d, d), lambda i: (0, 0))
    b_spec = pl.BlockSpec((1, d), lambda i: (0, 0))

    def body(app, aap, cpp, cap, xp, wpn, wan, wps, was, bpp, bap, out):
        inv_pp = 1.0 / jnp.maximum(jnp.sum(cpp[0], axis=0), 1.0)[:, None]
        inv_ap = 1.0 / jnp.maximum(jnp.sum(cap[0], axis=0), 1.0)[:, None]
        dot = functools.partial(jax.lax.dot,
                                precision=jax.lax.Precision.HIGHEST,
                                preferred_element_type=jnp.float32)
        acc = dot(app[0] * inv_pp, wpn[...])
        acc = acc + dot(aap[0] * inv_ap, wan[...])
        acc = acc + dot(xp[...], wps[...] + was[...])
        out[...] = acc + bpp[...] + bap[...]

    return pl.pallas_call(
        body,
        grid=grid,
        in_specs=[agg_spec, agg_spec2, cnt_spec, cnt_spec2, row_spec,
                  w_spec, w_spec, w_spec, w_spec, b_spec, b_spec],
        out_specs=row_spec,
        out_shape=jax.ShapeDtypeStruct((n, d), jnp.float32),
    )(agg, agg, cnt, cnt, x_paper,
      W_pp_nbr, W_ap_nbr, W_pp_self, W_ap_self,
      b_pp.reshape(1, d), b_ap.reshape(1, d))


def _pad_edges(ei, n_dst, chunks, src_offset):
    e = ei.shape[1]
    total = NUM_SUBCORES * chunks * CHUNK
    pad = total - e
    shape = (NUM_SUBCORES, chunks, CHUNK)
    src = jnp.concatenate(
        [ei[0] + src_offset, jnp.zeros((pad,), jnp.int32)]).reshape(shape)
    dst = jnp.concatenate(
        [ei[1], jnp.full((pad,), n_dst, jnp.int32)]).reshape(shape)
    return jnp.stack([src, dst], axis=1)


def kernel(x_paper, x_author, edge_index_paper_cites_paper,
           edge_index_author_writes_paper, W_pp_self, W_pp_nbr, b_pp,
           W_ap_self, W_ap_nbr, b_ap):
    n_dst = x_paper.shape[0]
    d = x_paper.shape[1]
    e = edge_index_paper_cites_paper.shape[1]
    chunks = BLK_CH * math.ceil(
        math.ceil(e / NUM_SUBCORES) / (CHUNK * BLK_CH))
    z_rows = 8 * math.ceil((n_dst + 1) / (8 * NUM_SUBCORES))

    x_flat = jnp.concatenate(
        [x_paper, x_author, jnp.zeros((z_rows, d), jnp.float32)], axis=0)
    idx_pp = _pad_edges(edge_index_paper_cites_paper, n_dst, chunks, 0)
    idx_ap = _pad_edges(edge_index_author_writes_paper, n_dst, chunks, n_dst)
    idx_all = jnp.stack([idx_pp, idx_ap], axis=0)
    idx_flat = idx_all.reshape(2, NUM_SUBCORES, 2, chunks * CHUNK)

    agg, cnt = _sc_segment_sums(x_flat, idx_all, idx_flat, n_dst, chunks,
                                z_rows)

    return _tc_combine(agg, cnt, x_paper,
                       W_pp_nbr, W_ap_nbr, W_pp_self, W_ap_self, b_pp, b_ap)


# speedup vs baseline: 1.0202x; 1.0202x over previous
import dataclasses
import functools
import math

import jax
import jax.numpy as jnp
from jax import lax
from jax.experimental import pallas as pl
from jax.experimental.pallas import tpu as pltpu
from jax.experimental.pallas import tpu_sc as plsc

NUM_CORES = 2
NUM_SUBCORES = 16
CHUNK = 128
LANES = 16
BLK_CH = 16
NBUF = 2


def _sc_segment_sums(x_flat, idx_all, idx_flat, n_dst, chunks, z_rows):
    d = x_flat.shape[1]
    zero_base = x_flat.shape[0] - z_rows
    sp_rows = NUM_SUBCORES * z_rows
    blocks = chunks // BLK_CH

    mesh = plsc.VectorSubcoreMesh(core_axis_name="c", subcore_axis_name="s")

    cp = pltpu.CompilerParams()
    if "needs_layout_passes" in pltpu.CompilerParams.__dataclass_fields__:
        cp = dataclasses.replace(cp, needs_layout_passes=False)

    @functools.partial(
        pl.kernel,
        compiler_params=cp,
        out_type=[
            jax.ShapeDtypeStruct((2, sp_rows, d), jnp.float32),
            jax.ShapeDtypeStruct((2, NUM_SUBCORES, sp_rows), jnp.float32),
        ],
        mesh=mesh,
        scratch_types=[
            pltpu.VMEM((2, BLK_CH, CHUNK), jnp.int32),
            pltpu.VMEM((BLK_CH * CHUNK,), jnp.int32),
            pltpu.VMEM((NBUF, CHUNK, d), jnp.float32),
            pltpu.VMEM((sp_rows,), jnp.float32),
            pltpu.VMEM_SHARED((sp_rows, d), jnp.float32),
            pltpu.SemaphoreType.DMA,
            pltpu.SemaphoreType.DMA,
        ],
    )
    def k(x_hbm, idx_hbm, idxf_hbm, agg_hbm, cnt_hbm,
          idx_v, dst_f, rows, cnt_l, agg_sh, sem0, sem1):
        c = lax.axis_index("c")
        s = lax.axis_index("s")
        sems = [sem0, sem1]

        @pl.loop(0, sp_rows // LANES)
        def _(i):
            cnt_l[pl.ds(i * LANES, LANES)] = jnp.zeros((LANES,), jnp.float32)

        pltpu.sync_copy(x_hbm.at[pl.ds(zero_base, z_rows)],
                        agg_sh.at[pl.ds(s * z_rows, z_rows)])
        plsc.subcore_barrier()

        ones = jnp.full((LANES,), 1.0, jnp.float32)

        @pl.loop(0, blocks)
        def _(b):
            pltpu.sync_copy(
                idx_hbm.at[c].at[s].at[:, pl.ds(b * BLK_CH, BLK_CH), :],
                idx_v)
            pltpu.sync_copy(
                idxf_hbm.at[c].at[s].at[1].at[
                    pl.ds(b * BLK_CH * CHUNK, BLK_CH * CHUNK)],
                dst_f)

            for t in range(NBUF):
                pltpu.async_copy(x_hbm.at[idx_v.at[0].at[t]], rows.at[t],
                                 sems[t])

            @pl.loop(0, BLK_CH // NBUF)
            def _(g):
                for t in range(NBUF):
                    j = g * NBUF + t
                    pltpu.make_async_copy(x_hbm.at[idx_v.at[0].at[j]],
                                          rows.at[t], sems[t]).wait()
                    pltpu.sync_copy(rows.at[t],
                                    agg_sh.at[idx_v.at[1].at[j]], add=True)
                    for l in range(CHUNK // LANES):
                        idx16 = dst_f[pl.ds(j * CHUNK + l * LANES, LANES)]
                        plsc.addupdate_scatter(cnt_l, [idx16], ones)
                    @pl.when(g < BLK_CH // NBUF - 1)
                    def _():
                        pltpu.async_copy(
                            x_hbm.at[idx_v.at[0].at[j + NBUF]],
                            rows.at[t], sems[t])

        plsc.subcore_barrier()

        pltpu.sync_copy(agg_sh.at[pl.ds(s * z_rows, z_rows)],
                        agg_hbm.at[c].at[pl.ds(s * z_rows, z_rows)])
        pltpu.sync_copy(cnt_l, cnt_hbm.at[c].at[s])

    return k(x_flat, idx_all, idx_flat)


def _tc_combine(agg, cnt, x_paper,
                W_pp_nbr, W_ap_nbr, W_pp_self, W_ap_self, b_pp, b_ap):
    n, d = x_paper.shape
    blk = 1024
    grid = (math.ceil(n / blk),)
    agg_spec = pl.BlockSpec((1, blk, d), lambda i: (0, i, 0))
    agg_spec2 = pl.BlockSpec((1, blk, d), lambda i: (1, i, 0))
    cnt_spec = pl.BlockSpec((1, NUM_SUBCORES, blk), lambda i: (0, 0, i))
    cnt_spec2 = pl.BlockSpec((1, NUM_SUBCORES, blk), lambda i: (1, 0, i))
    row_spec = pl.BlockSpec((blk, d), lambda i: (i, 0))
    w_spec = pl.BlockSpec((d, d), lambda i: (0, 0))
    b_spec = pl.BlockSpec((1, d), lambda i: (0, 0))

    def body(app, aap, cpp, cap, xp, wpn, wan, wps, was, bpp, bap, out):
        inv_pp = 1.0 / jnp.maximum(jnp.sum(cpp[0], axis=0), 1.0)[:, None]
        inv_ap = 1.0 / jnp.maximum(jnp.sum(cap[0], axis=0), 1.0)[:, None]
        dot = functools.partial(jax.lax.dot,
                                precision=jax.lax.Precision.HIGHEST,
                                preferred_element_type=jnp.float32)
        acc = dot(app[0] * inv_pp, wpn[...])
        acc = acc + dot(aap[0] * inv_ap, wan[...])
        acc = acc + dot(xp[...], wps[...] + was[...])
        out[...] = acc + bpp[...] + bap[...]

    return pl.pallas_call(
        body,
        grid=grid,
        in_specs=[agg_spec, agg_spec2, cnt_spec, cnt_spec2, row_spec,
                  w_spec, w_spec, w_spec, w_spec, b_spec, b_spec],
        out_specs=row_spec,
        out_shape=jax.ShapeDtypeStruct((n, d), jnp.float32),
    )(agg, agg, cnt, cnt, x_paper,
      W_pp_nbr, W_ap_nbr, W_pp_self, W_ap_self,
      b_pp.reshape(1, d), b_ap.reshape(1, d))


def _pad_edges(ei, n_dst, chunks, src_offset):
    e = ei.shape[1]
    total = NUM_SUBCORES * chunks * CHUNK
    pad = total - e
    shape = (NUM_SUBCORES, chunks, CHUNK)
    src = jnp.concatenate(
        [ei[0] + src_offset, jnp.zeros((pad,), jnp.int32)]).reshape(shape)
    dst = jnp.concatenate(
        [ei[1], jnp.full((pad,), n_dst, jnp.int32)]).reshape(shape)
    return jnp.stack([src, dst], axis=1)


def kernel(x_paper, x_author, edge_index_paper_cites_paper,
           edge_index_author_writes_paper, W_pp_self, W_pp_nbr, b_pp,
           W_ap_self, W_ap_nbr, b_ap):
    n_dst = x_paper.shape[0]
    d = x_paper.shape[1]
    e = edge_index_paper_cites_paper.shape[1]
    chunks = BLK_CH * math.ceil(
        math.ceil(e / NUM_SUBCORES) / (CHUNK * BLK_CH))
    z_rows = 8 * math.ceil((n_dst + 1) / (8 * NUM_SUBCORES))

    x_flat = jnp.concatenate(
        [x_paper, x_author, jnp.zeros((z_rows, d), jnp.float32)], axis=0)
    idx_pp = _pad_edges(edge_index_paper_cites_paper, n_dst, chunks, 0)
    idx_ap = _pad_edges(edge_index_author_writes_paper, n_dst, chunks, n_dst)
    idx_all = jnp.stack([idx_pp, idx_ap], axis=0)
    idx_flat = idx_all.reshape(2, NUM_SUBCORES, 2, chunks * CHUNK)

    agg, cnt = _sc_segment_sums(x_flat, idx_all, idx_flat, n_dst, chunks,
                                z_rows)

    return _tc_combine(agg, cnt, x_paper,
                       W_pp_nbr, W_ap_nbr, W_pp_self, W_ap_self, b_pp, b_ap)


# speedup vs baseline: 1.0224x; 1.0022x over previous
import dataclasses
import functools
import math

import jax
import jax.numpy as jnp
from jax import lax
from jax.experimental import pallas as pl
from jax.experimental.pallas import tpu as pltpu
from jax.experimental.pallas import tpu_sc as plsc

NUM_CORES = 2
NUM_SUBCORES = 16
CHUNK = 128
LANES = 16
BLK_CH = 16
NBUF = 2


def _sc_segment_sums(x_flat, idx_all, idx_flat, n_dst, chunks, z_rows):
    d = x_flat.shape[1]
    zero_base = x_flat.shape[0] - z_rows
    sp_rows = NUM_SUBCORES * z_rows
    blocks = chunks // BLK_CH

    mesh = plsc.VectorSubcoreMesh(core_axis_name="c", subcore_axis_name="s")

    cp = pltpu.CompilerParams()
    if "needs_layout_passes" in pltpu.CompilerParams.__dataclass_fields__:
        cp = dataclasses.replace(cp, needs_layout_passes=False)

    @functools.partial(
        pl.kernel,
        compiler_params=cp,
        out_type=[
            jax.ShapeDtypeStruct((2, sp_rows, d), jnp.float32),
            jax.ShapeDtypeStruct((2, NUM_SUBCORES, sp_rows), jnp.float32),
        ],
        mesh=mesh,
        scratch_types=[
            pltpu.VMEM((2, BLK_CH, CHUNK), jnp.int32),
            pltpu.VMEM((BLK_CH * CHUNK,), jnp.int32),
            pltpu.VMEM((NBUF, CHUNK, d), jnp.float32),
            pltpu.VMEM((sp_rows,), jnp.float32),
            pltpu.VMEM_SHARED((sp_rows, d), jnp.float32),
            pltpu.SemaphoreType.DMA,
            pltpu.SemaphoreType.DMA,
        ],
    )
    def k(x_hbm, idx_hbm, idxf_hbm, agg_hbm, cnt_hbm,
          idx_v, dst_f, rows, cnt_l, agg_sh, sem0, sem1):
        c = lax.axis_index("c")
        s = lax.axis_index("s")
        sems = [sem0, sem1]

        @pl.loop(0, sp_rows // LANES)
        def _(i):
            cnt_l[pl.ds(i * LANES, LANES)] = jnp.zeros((LANES,), jnp.float32)

        pltpu.sync_copy(x_hbm.at[pl.ds(zero_base, z_rows)],
                        agg_sh.at[pl.ds(s * z_rows, z_rows)])
        plsc.subcore_barrier()

        ones = jnp.full((LANES,), 1.0, jnp.float32)

        @pl.loop(0, blocks)
        def _(b):
            pltpu.sync_copy(
                idx_hbm.at[c].at[s].at[:, pl.ds(b * BLK_CH, BLK_CH), :],
                idx_v)
            pltpu.sync_copy(
                idxf_hbm.at[c].at[s].at[1].at[
                    pl.ds(b * BLK_CH * CHUNK, BLK_CH * CHUNK)],
                dst_f)

            for t in range(NBUF):
                pltpu.async_copy(x_hbm.at[idx_v.at[0].at[t]], rows.at[t],
                                 sems[t])

            @pl.loop(0, BLK_CH // NBUF)
            def _(g):
                for t in range(NBUF):
                    j = g * NBUF + t
                    pltpu.make_async_copy(x_hbm.at[idx_v.at[0].at[j]],
                                          rows.at[t], sems[t]).wait()
                    pltpu.sync_copy(rows.at[t],
                                    agg_sh.at[idx_v.at[1].at[j]], add=True)
                    @pl.when(g < BLK_CH // NBUF - 1)
                    def _():
                        pltpu.async_copy(
                            x_hbm.at[idx_v.at[0].at[j + NBUF]],
                            rows.at[t], sems[t])
                    for l in range(CHUNK // LANES):
                        idx16 = dst_f[pl.ds(j * CHUNK + l * LANES, LANES)]
                        plsc.addupdate_scatter(cnt_l, [idx16], ones)

        plsc.subcore_barrier()

        pltpu.sync_copy(agg_sh.at[pl.ds(s * z_rows, z_rows)],
                        agg_hbm.at[c].at[pl.ds(s * z_rows, z_rows)])
        pltpu.sync_copy(cnt_l, cnt_hbm.at[c].at[s])

    return k(x_flat, idx_all, idx_flat)


def _tc_combine(agg, cnt, x_paper,
                W_pp_nbr, W_ap_nbr, W_pp_self, W_ap_self, b_pp, b_ap):
    n, d = x_paper.shape
    blk = 1024
    grid = (math.ceil(n / blk),)
    agg_spec = pl.BlockSpec((1, blk, d), lambda i: (0, i, 0))
    agg_spec2 = pl.BlockSpec((1, blk, d), lambda i: (1, i, 0))
    cnt_spec = pl.BlockSpec((1, NUM_SUBCORES, blk), lambda i: (0, 0, i))
    cnt_spec2 = pl.BlockSpec((1, NUM_SUBCORES, blk), lambda i: (1, 0, i))
    row_spec = pl.BlockSpec((blk, d), lambda i: (i, 0))
    w_spec = pl.BlockSpec((d, d), lambda i: (0, 0))
    b_spec = pl.BlockSpec((1, d), lambda i: (0, 0))

    def body(app, aap, cpp, cap, xp, wpn, wan, wps, was, bpp, bap, out):
        inv_pp = 1.0 / jnp.maximum(jnp.sum(cpp[0], axis=0), 1.0)[:, None]
        inv_ap = 1.0 / jnp.maximum(jnp.sum(cap[0], axis=0), 1.0)[:, None]
        dot = functools.partial(jax.lax.dot,
                                precision=jax.lax.Precision.HIGHEST,
                                preferred_element_type=jnp.float32)
        acc = dot(app[0] * inv_pp, wpn[...])
        acc = acc + dot(aap[0] * inv_ap, wan[...])
        acc = acc + dot(xp[...], wps[...] + was[...])
        out[...] = acc + bpp[...] + bap[...]

    return pl.pallas_call(
        body,
        grid=grid,
        in_specs=[agg_spec, agg_spec2, cnt_spec, cnt_spec2, row_spec,
                  w_spec, w_spec, w_spec, w_spec, b_spec, b_spec],
        out_specs=row_spec,
        out_shape=jax.ShapeDtypeStruct((n, d), jnp.float32),
    )(agg, agg, cnt, cnt, x_paper,
      W_pp_nbr, W_ap_nbr, W_pp_self, W_ap_self,
      b_pp.reshape(1, d), b_ap.reshape(1, d))


def _pad_edges(ei, n_dst, chunks, src_offset):
    e = ei.shape[1]
    total = NUM_SUBCORES * chunks * CHUNK
    pad = total - e
    shape = (NUM_SUBCORES, chunks, CHUNK)
    src = jnp.concatenate(
        [ei[0] + src_offset, jnp.zeros((pad,), jnp.int32)]).reshape(shape)
    dst = jnp.concatenate(
        [ei[1], jnp.full((pad,), n_dst, jnp.int32)]).reshape(shape)
    return jnp.stack([src, dst], axis=1)


def kernel(x_paper, x_author, edge_index_paper_cites_paper,
           edge_index_author_writes_paper, W_pp_self, W_pp_nbr, b_pp,
           W_ap_self, W_ap_nbr, b_ap):
    n_dst = x_paper.shape[0]
    d = x_paper.shape[1]
    e = edge_index_paper_cites_paper.shape[1]
    chunks = BLK_CH * math.ceil(
        math.ceil(e / NUM_SUBCORES) / (CHUNK * BLK_CH))
    z_rows = 8 * math.ceil((n_dst + 1) / (8 * NUM_SUBCORES))

    x_flat = jnp.concatenate(
        [x_paper, x_author, jnp.zeros((z_rows, d), jnp.float32)], axis=0)
    idx_pp = _pad_edges(edge_index_paper_cites_paper, n_dst, chunks, 0)
    idx_ap = _pad_edges(edge_index_author_writes_paper, n_dst, chunks, n_dst)
    idx_all = jnp.stack([idx_pp, idx_ap], axis=0)
    idx_flat = idx_all.reshape(2, NUM_SUBCORES, 2, chunks * CHUNK)

    agg, cnt = _sc_segment_sums(x_flat, idx_all, idx_flat, n_dst, chunks,
                                z_rows)

    return _tc_combine(agg, cnt, x_paper,
                       W_pp_nbr, W_ap_nbr, W_pp_self, W_ap_self, b_pp, b_ap)
